# baseline (speedup 1.0000x reference)
import functools
import jax
import jax.numpy as jnp
from jax import lax
from jax.experimental import pallas as pl
from jax.experimental.pallas import tpu as pltpu
from jax.experimental.pallas import tpu_sc as plsc

N = 100000
E = 6400000
NC = 2
NS = 16
NW = NC * NS
L = 16
NBUF = 4
CHUNK = 6400
NCH = E // CHUNK
SUBIT = (NCH + NW - 1) // NW + 2
MACRO = (SUBIT + NBUF - 1) // NBUF
UNROLL = 4
NPT = 6272
NPAD = NS * NPT
VSLICE = 6256

_mesh = plsc.VectorSubcoreMesh(
    core_axis_name="c", subcore_axis_name="s", num_cores=NC)


@functools.partial(
    pl.kernel,
    out_type=jax.ShapeDtypeStruct((NC, NPAD), jnp.float32),
    mesh=_mesh,
    scratch_types=[
        pltpu.VMEM((CHUNK,), jnp.int32),
        pltpu.VMEM((CHUNK,), jnp.int32),
        pltpu.VMEM((CHUNK,), jnp.int32),
        pltpu.VMEM((CHUNK,), jnp.int32),
        pltpu.VMEM((CHUNK,), jnp.int32),
        pltpu.VMEM((CHUNK,), jnp.int32),
        pltpu.VMEM((CHUNK,), jnp.int32),
        pltpu.VMEM((CHUNK,), jnp.int32),
        pltpu.VMEM((CHUNK,), jnp.float32),
        pltpu.VMEM((CHUNK,), jnp.float32),
        pltpu.VMEM((CHUNK,), jnp.float32),
        pltpu.VMEM((CHUNK,), jnp.float32),
        pltpu.VMEM((CHUNK,), jnp.float32),
        pltpu.VMEM((CHUNK,), jnp.float32),
        pltpu.VMEM((CHUNK,), jnp.float32),
        pltpu.VMEM((CHUNK,), jnp.float32),
        pltpu.VMEM_SHARED((N,), jnp.float32),
        pltpu.VMEM_SHARED((NPAD,), jnp.float32),
        pltpu.SemaphoreType.DMA((NBUF,)),
        pltpu.SemaphoreType.DMA((NBUF,)),
        pltpu.SemaphoreType.DMA((NBUF,)),
    ],
    compiler_params=pltpu.CompilerParams(needs_layout_passes=False),
)
def _sc_scatter(eif_hbm, w_hbm, vals_hbm, out_hbm,
                src_a, src_b, src_c, src_d, dst_a, dst_b, dst_c, dst_d,
                wm_a, wm_b, wm_c, wm_d, gb_a, gb_b, gb_c, gb_d,
                vals_sh, acc_sh, sem_in, sem_g, sem_sc):
    srcs = (src_a, src_b, src_c, src_d)
    dsts = (dst_a, dst_b, dst_c, dst_d)
    wms = (wm_a, wm_b, wm_c, wm_d)
    gbs = (gb_a, gb_b, gb_c, gb_d)
    cid = lax.axis_index("c")
    sid = lax.axis_index("s")
    wid = sid * NC + cid

    def _fire_in(c, j):
        base = pl.multiple_of(c * CHUNK, 8)
        pltpu.async_copy(eif_hbm.at[0, pl.ds(base, CHUNK)], srcs[j],
                         sem_in.at[j])
        pltpu.async_copy(eif_hbm.at[1, pl.ds(base, CHUNK)], dsts[j],
                         sem_in.at[j])
        pltpu.async_copy(w_hbm.at[pl.ds(base, CHUNK)], wms[j],
                         sem_in.at[j])

    def _wait_in(c, j):
        base = pl.multiple_of(c * CHUNK, 8)
        pltpu.make_async_copy(eif_hbm.at[0, pl.ds(base, CHUNK)], srcs[j],
                              sem_in.at[j]).wait()
        pltpu.make_async_copy(eif_hbm.at[1, pl.ds(base, CHUNK)], dsts[j],
                              sem_in.at[j]).wait()
        pltpu.make_async_copy(w_hbm.at[pl.ds(base, CHUNK)], wms[j],
                              sem_in.at[j]).wait()

    vbase = sid * VSLICE

    @pl.when(sid < NS - 1)
    def _():
        pltpu.sync_copy(vals_hbm.at[pl.ds(vbase, VSLICE)],
                        gb_a.at[pl.ds(0, VSLICE)])
        pltpu.sync_copy(gb_a.at[pl.ds(0, VSLICE)],
                        vals_sh.at[pl.ds(vbase, VSLICE)])

    VTAIL = N - (NS - 1) * VSLICE

    @pl.when(sid == NS - 1)
    def _():
        pltpu.sync_copy(vals_hbm.at[pl.ds((NS - 1) * VSLICE, VTAIL)],
                        gb_a.at[pl.ds(0, VTAIL)])
        pltpu.sync_copy(gb_a.at[pl.ds(0, VTAIL)],
                        vals_sh.at[pl.ds((NS - 1) * VSLICE, VTAIL)])

    zeros = jnp.zeros((L,), jnp.float32)

    def _z(i, carry):
        gb_a[pl.ds(i * L, L)] = zeros
        return carry

    lax.fori_loop(0, CHUNK // L, _z, 0)
    for k in range(NPT // CHUNK):
        pltpu.sync_copy(gb_a,
                        acc_sh.at[pl.ds(sid * NPT + k * CHUNK, CHUNK)])
    pltpu.sync_copy(gb_a.at[pl.ds(0, NPT % CHUNK)],
                    acc_sh.at[pl.ds(sid * NPT + NPT - NPT % CHUNK,
                                    NPT % CHUNK)])

    _fire_in(wid, 0)
    _fire_in(NW + wid, 1)

    plsc.subcore_barrier()

    def _macro(i, carry):
        for j in range(NBUF):
            i3 = i * NBUF + j
            c = i3 * NW + wid

            @pl.when(c < NCH)
            def _():
                _wait_in(c, j)
                pltpu.async_copy(vals_sh.at[srcs[j]], gbs[j], sem_g.at[j])

            jm1 = (j + NBUF - 1) % NBUF
            cm1 = c - NW

            @pl.when((i3 >= 1) & (cm1 < NCH))
            def _():
                pltpu.make_async_copy(vals_sh.at[srcs[jm1]], gbs[jm1],
                                      sem_g.at[jm1]).wait()

                def _g(g, acc):
                    for u in range(UNROLL):
                        o = g * (L * UNROLL) + u * L
                        wms[jm1][pl.ds(o, L)] = (
                            gbs[jm1][pl.ds(o, L)] * wms[jm1][pl.ds(o, L)])
                    return acc

                lax.fori_loop(0, CHUNK // (L * UNROLL), _g, 0)
                pltpu.async_copy(wms[jm1], acc_sh.at[dsts[jm1]],
                                 sem_sc.at[jm1], add=True)

            jm2 = (j + NBUF - 2) % NBUF
            cm2 = c - 2 * NW

            @pl.when((i3 >= 2) & (cm2 < NCH))
            def _():
                pltpu.make_async_copy(wms[jm2], acc_sh.at[dsts[jm2]],
                                      sem_sc.at[jm2]).wait()

            cf = c + 2 * NW

            @pl.when(cf < NCH)
            def _():
                _fire_in(cf, jm2)

        return carry

    lax.fori_loop(0, MACRO, _macro, 0)

    plsc.subcore_barrier()
    pltpu.sync_copy(acc_sh.at[pl.ds(sid * NPT, NPT)],
                    out_hbm.at[cid, pl.ds(sid * NPT, NPT)])


def _combine_body(x_ref, o_ref):
    s = jax.nn.sigmoid(x_ref[0] + x_ref[1])
    o_ref[...] = s[:N]


_combine = pl.pallas_call(
    _combine_body,
    out_shape=jax.ShapeDtypeStruct((N,), jnp.float32),
)


@jax.jit
def kernel(values, edge_index, edge_weight):
    partials = _sc_scatter(edge_index, edge_weight, values)
    return _combine(partials)
